# baseline (speedup 1.0000x reference)
import functools

import jax
import jax.numpy as jnp
from jax.experimental import pallas as pl
from jax.experimental.pallas import tpu as pltpu

B = 4096
G = 19264
D = 64
BM = 32
NCHUNK = B // BM
NBUF = 16


def _body(x_hbm, emb_hbm, out_ref, xbuf, embv, dsems, esem):
    def start_chunk(chunk, slot):
        pltpu.make_async_copy(
            x_hbm.at[pl.ds(chunk * BM, BM), :], xbuf.at[slot], dsems.at[slot]
        ).start()

    pltpu.make_async_copy(emb_hbm, embv, esem).start()
    for j in range(NBUF):
        start_chunk(j, j)
    pltpu.make_async_copy(emb_hbm, embv, esem).wait()

    def step(i, carry):
        slot = jax.lax.rem(i, NBUF)
        pltpu.make_async_copy(
            x_hbm.at[pl.ds(i * BM, BM), :], xbuf.at[slot], dsems.at[slot]
        ).wait()
        out_ref[pl.ds(i * BM, BM), :] = jax.lax.dot_general(
            xbuf[slot], embv[...],
            dimension_numbers=(((1,), (0,)), ((), ())),
            preferred_element_type=jnp.float32,
        )
        nxt = i + NBUF

        @pl.when(nxt < NCHUNK)
        def _():
            start_chunk(nxt, slot)

        return carry

    jax.lax.fori_loop(0, NCHUNK, step, 0)


@functools.partial(jax.jit, static_argnames=())
def _feat(x, emb):
    return pl.pallas_call(
        _body,
        in_specs=[
            pl.BlockSpec(memory_space=pltpu.MemorySpace.HBM),
            pl.BlockSpec(memory_space=pltpu.MemorySpace.HBM),
        ],
        out_specs=pl.BlockSpec(memory_space=pltpu.MemorySpace.VMEM),
        out_shape=jax.ShapeDtypeStruct((B, D), jnp.float32),
        scratch_shapes=[
            pltpu.VMEM((NBUF, BM, G), jnp.float32),
            pltpu.VMEM((G, D), jnp.float32),
            pltpu.SemaphoreType.DMA((NBUF,)),
            pltpu.SemaphoreType.DMA,
        ],
        compiler_params=pltpu.CompilerParams(
            vmem_limit_bytes=100 * 1024 * 1024,
        ),
    )(x, emb)


def kernel(x, emb):
    feat = _feat(x, emb)
    return (feat, emb)


# speedup vs baseline: 1.0132x; 1.0132x over previous
import functools

import jax
import jax.numpy as jnp
from jax.experimental import pallas as pl
from jax.experimental.pallas import tpu as pltpu

B = 4096
G = 19264
D = 64
BM = 32
NCHUNK = B // BM
NBUF = 16


def _body(x_hbm, emb_hbm, out_ref, xbuf, embv, dsems, esem):
    def start_chunk(chunk, slot, prio=0):
        pltpu.make_async_copy(
            x_hbm.at[pl.ds(chunk * BM, BM), :], xbuf.at[slot], dsems.at[slot]
        ).start(priority=prio)

    pltpu.make_async_copy(emb_hbm, embv, esem).start()
    for j in range(NBUF):
        start_chunk(j, j, j % 2)
    pltpu.make_async_copy(emb_hbm, embv, esem).wait()

    def step(i, carry):
        slot = jax.lax.rem(i, NBUF)
        pltpu.make_async_copy(
            x_hbm.at[pl.ds(i * BM, BM), :], xbuf.at[slot], dsems.at[slot]
        ).wait()
        out_ref[pl.ds(i * BM, BM), :] = jax.lax.dot_general(
            xbuf[slot], embv[...],
            dimension_numbers=(((1,), (0,)), ((), ())),
            preferred_element_type=jnp.float32,
        )
        nxt = i + NBUF
        even = jax.lax.rem(slot, 2) == 0

        @pl.when(jnp.logical_and(nxt < NCHUNK, even))
        def _():
            start_chunk(nxt, slot, 0)

        @pl.when(jnp.logical_and(nxt < NCHUNK, jnp.logical_not(even)))
        def _():
            start_chunk(nxt, slot, 1)

        return carry

    jax.lax.fori_loop(0, NCHUNK, step, 0)


@functools.partial(jax.jit, static_argnames=())
def _feat(x, emb):
    return pl.pallas_call(
        _body,
        in_specs=[
            pl.BlockSpec(memory_space=pltpu.MemorySpace.HBM),
            pl.BlockSpec(memory_space=pltpu.MemorySpace.HBM),
        ],
        out_specs=pl.BlockSpec(memory_space=pltpu.MemorySpace.VMEM),
        out_shape=jax.ShapeDtypeStruct((B, D), jnp.float32),
        scratch_shapes=[
            pltpu.VMEM((NBUF, BM, G), jnp.float32),
            pltpu.VMEM((G, D), jnp.float32),
            pltpu.SemaphoreType.DMA((NBUF,)),
            pltpu.SemaphoreType.DMA,
        ],
        compiler_params=pltpu.CompilerParams(
            vmem_limit_bytes=100 * 1024 * 1024,
        ),
    )(x, emb)


def kernel(x, emb):
    feat = _feat(x, emb)
    return (feat, emb)


# speedup vs baseline: 1.0526x; 1.0390x over previous
import functools

import jax
import jax.numpy as jnp
from jax.experimental import pallas as pl
from jax.experimental.pallas import tpu as pltpu

B = 4096
G = 19264
D = 64
BM = 256


def _matmul_body(x_ref, emb_ref, out_ref):
    out_ref[...] = jax.lax.dot_general(
        x_ref[...], emb_ref[...],
        dimension_numbers=(((1,), (0,)), ((), ())),
        preferred_element_type=jnp.float32,
    )


@functools.partial(jax.jit, static_argnames=())
def _feat(x, emb):
    grid = (B // BM,)
    return pl.pallas_call(
        _matmul_body,
        grid=grid,
        in_specs=[
            pl.BlockSpec((BM, G), lambda i: (i, 0)),
            pl.BlockSpec((G, D), lambda i: (0, 0)),
        ],
        out_specs=pl.BlockSpec((BM, D), lambda i: (i, 0)),
        out_shape=jax.ShapeDtypeStruct((B, D), jnp.float32),
    )(x, emb)


def kernel(x, emb):
    feat = _feat(x, emb)
    return (feat, emb)
